# baseline (speedup 1.0000x reference)
import dataclasses
import functools

import jax
import jax.numpy as jnp
from jax import lax
from jax.experimental import pallas as pl
from jax.experimental.pallas import tpu as pltpu
from jax.experimental.pallas import tpu_sc as plsc

EPS = 1e-8
LANES = 16
NUM_CORES = 2
NUM_SUBCORES = 16
NW = NUM_CORES * NUM_SUBCORES
CHUNK = 64
NBUF = 3


def _normalize_body(x_ref, o_ref):
    x = x_ref[...]
    ss = jnp.sum(x * x, axis=1, keepdims=True)
    inv = 1.0 / jnp.maximum(jnp.sqrt(ss), EPS)
    o_ref[...] = x * inv


def _finalize_body(s_ref, d_ref, o_ref):
    s = jnp.sum(s_ref[...], axis=0, keepdims=True)
    d = jnp.sum(d_ref[...], axis=0, keepdims=True)
    o_ref[...] = jnp.where(d > 0.0, s / jnp.maximum(d, 1.0), 1.0)


@functools.lru_cache(maxsize=None)
def _make_edge_kernel(n_nodes, d, ch, n_edges):
    nseg = d // LANES
    n_pad = -(-n_nodes // 2048) * 2048
    mesh = plsc.VectorSubcoreMesh(core_axis_name="c", subcore_axis_name="s")
    out_t = (
        jax.ShapeDtypeStruct((NUM_CORES, n_pad), jnp.float32),
        jax.ShapeDtypeStruct((NUM_CORES, n_pad), jnp.float32),
    )

    cp = pltpu.CompilerParams()
    if "needs_layout_passes" in pltpu.CompilerParams.__dataclass_fields__:
        cp = dataclasses.replace(cp, needs_layout_passes=False)

    @functools.partial(
        pl.kernel,
        out_type=out_t,
        mesh=mesh,
        compiler_params=cp,
        scratch_types=[
            pltpu.VMEM((ch, CHUNK), jnp.int32),
            pltpu.VMEM((ch, CHUNK), jnp.int32),
            pltpu.VMEM((NBUF, CHUNK), jnp.float32),
            pltpu.VMEM((NBUF, CHUNK, d), jnp.float32),
            pltpu.VMEM((NBUF, CHUNK, d), jnp.float32),
            pltpu.VMEM((NBUF, CHUNK), jnp.float32),
            pltpu.VMEM((LANES * LANES,), jnp.float32),
            pltpu.VMEM((2048,), jnp.float32),
            pltpu.VMEM_SHARED((n_pad,), jnp.float32),
            pltpu.VMEM_SHARED((n_pad,), jnp.float32),
        ]
        + [pltpu.SemaphoreType.DMA] * (3 * NBUF),
    )
    def edge_kernel(xhat_hbm, src_hbm, dst_hbm, sums_hbm, degs_hbm,
                    src_v, dst_v, val_c, srows, drows, sim_v, tmp_v, stage_v,
                    shared_sum, shared_deg, *sems):
        sem_gs = sems[:NBUF]
        sem_gd = sems[NBUF:2 * NBUF]
        sem_sc = sems[2 * NBUF:]
        cid = lax.axis_index("c")
        sid = lax.axis_index("s")
        wid = sid * NUM_CORES + cid
        zeros16 = jnp.zeros((LANES,), jnp.float32)
        lane_iota = lax.iota(jnp.int32, LANES)

        @pl.when(sid == 0)
        def _init():
            @pl.loop(0, 2048, step=LANES)
            def _z(i):
                stage_v[pl.ds(pl.multiple_of(i, LANES), LANES)] = zeros16

            @pl.loop(0, n_pad, step=2048)
            def _zs(i):
                ii = pl.multiple_of(i, 2048)
                pltpu.sync_copy(stage_v, shared_sum.at[pl.ds(ii, 2048)])
                pltpu.sync_copy(stage_v, shared_deg.at[pl.ds(ii, 2048)])

        pltpu.sync_copy(src_hbm.at[wid], src_v)
        pltpu.sync_copy(dst_hbm.at[wid], dst_v)

        for b in range(NBUF):
            pltpu.async_copy(xhat_hbm.at[src_v.at[b]], srows.at[b], sem_gs[b])
            pltpu.async_copy(xhat_hbm.at[dst_v.at[b]], drows.at[b], sem_gd[b])
        plsc.subcore_barrier()

        def _scatter_descs(b, j):
            return (
                pltpu.make_async_copy(
                    sim_v.at[b], shared_sum.at[src_v.at[j]], sem_sc[b]),
                pltpu.make_async_copy(
                    sim_v.at[b], shared_sum.at[dst_v.at[j]], sem_sc[b]),
                pltpu.make_async_copy(
                    val_c.at[b], shared_deg.at[src_v.at[j]], sem_sc[b]),
                pltpu.make_async_copy(
                    val_c.at[b], shared_deg.at[dst_v.at[j]], sem_sc[b]),
            )

        @pl.loop(0, ch, step=NBUF)
        def _ring(g):
            for b in range(NBUF):
                j = g + b
                srow = srows.at[b]
                drow = drows.at[b]
                pltpu.make_async_copy(xhat_hbm.at[src_v.at[j]], srow, sem_gs[b]).wait()
                pltpu.make_async_copy(xhat_hbm.at[dst_v.at[j]], drow, sem_gd[b]).wait()
                @pl.when(j >= NBUF)
                def _drain_scatters():
                    for cpd in _scatter_descs(b, j - NBUF):
                        cpd.wait()

                chunk_eid = (wid * ch + j) * CHUNK

                @pl.loop(0, CHUNK // LANES)
                def _group(g2):
                    base = pl.multiple_of(g2 * LANES, LANES)
                    vmask = jnp.where(chunk_eid + base + lane_iota < n_edges,
                                      1.0, 0.0).astype(jnp.float32)
                    for rr in range(LANES):
                        a = srow[base + rr, pl.ds(0, LANES)]
                        bb = drow[base + rr, pl.ds(0, LANES)]
                        acc = a * bb
                        for kk in range(1, nseg):
                            a = srow[base + rr, pl.ds(kk * LANES, LANES)]
                            bb = drow[base + rr, pl.ds(kk * LANES, LANES)]
                            acc = acc + a * bb
                        tmp_v[pl.ds(rr * LANES, LANES)] = acc
                    col = zeros16
                    for c in range(LANES):
                        rot = jnp.bitwise_and(lane_iota + c, LANES - 1)
                        col = col + plsc.load_gather(
                            tmp_v, [lane_iota * LANES + rot])
                    sim_v[b, pl.ds(base, LANES)] = col * vmask
                    val_c[b, pl.ds(base, LANES)] = vmask

                pltpu.async_copy(sim_v.at[b], shared_sum.at[src_v.at[j]],
                                 sem_sc[b], add=True)
                pltpu.async_copy(sim_v.at[b], shared_sum.at[dst_v.at[j]],
                                 sem_sc[b], add=True)
                pltpu.async_copy(val_c.at[b], shared_deg.at[src_v.at[j]],
                                 sem_sc[b], add=True)
                pltpu.async_copy(val_c.at[b], shared_deg.at[dst_v.at[j]],
                                 sem_sc[b], add=True)

                @pl.when(j + NBUF < ch)
                def _prefetch():
                    pltpu.async_copy(xhat_hbm.at[src_v.at[j + NBUF]], srows.at[b], sem_gs[b])
                    pltpu.async_copy(xhat_hbm.at[dst_v.at[j + NBUF]], drows.at[b], sem_gd[b])

        for b in range(NBUF):
            for cpd in _scatter_descs(b, ch - NBUF + b):
                cpd.wait()

        plsc.subcore_barrier()

        @pl.when(sid == 0)
        def _drain():
            @pl.loop(0, n_pad, step=2048)
            def _d(i):
                ii = pl.multiple_of(i, 2048)
                pltpu.sync_copy(shared_sum.at[pl.ds(ii, 2048)], stage_v)
                pltpu.sync_copy(stage_v, sums_hbm.at[cid].at[pl.ds(ii, 2048)])
                pltpu.sync_copy(shared_deg.at[pl.ds(ii, 2048)], stage_v)
                pltpu.sync_copy(stage_v, degs_hbm.at[cid].at[pl.ds(ii, 2048)])

    return edge_kernel


def kernel(node_features, edge_index):
    n, d = node_features.shape
    e = edge_index.shape[1]

    xhat = pl.pallas_call(
        _normalize_body,
        out_shape=jax.ShapeDtypeStruct((n, d), jnp.float32),
    )(node_features)

    ch = -(-e // (NW * CHUNK))
    ch = NBUF * (-(-ch // NBUF))
    ep = NW * CHUNK * ch
    pad = ep - e
    src = edge_index[0].astype(jnp.int32)
    dst = edge_index[1].astype(jnp.int32)
    srcp = jnp.pad(src, (0, pad)).reshape(NW, ch, CHUNK)
    dstp = jnp.pad(dst, (0, pad)).reshape(NW, ch, CHUNK)

    sums, degs = _make_edge_kernel(n, d, ch, e)(xhat, srcp, dstp)

    n_pad = sums.shape[1]
    out = pl.pallas_call(
        _finalize_body,
        out_shape=jax.ShapeDtypeStruct((1, n_pad), jnp.float32),
    )(sums, degs)
    return out.reshape(n_pad)[:n]


# speedup vs baseline: 1.5701x; 1.5701x over previous
import dataclasses
import functools

import jax
import jax.numpy as jnp
from jax import lax
from jax.experimental import pallas as pl
from jax.experimental.pallas import tpu as pltpu
from jax.experimental.pallas import tpu_sc as plsc

EPS = 1e-8
LANES = 16
NUM_CORES = 2
NUM_SUBCORES = 16
NW = NUM_CORES * NUM_SUBCORES
CHUNK = 64


def _normalize_body(x_ref, o_ref):
    x = x_ref[...]
    ss = jnp.sum(x * x, axis=1, keepdims=True)
    inv = 1.0 / jnp.maximum(jnp.sqrt(ss), EPS)
    o_ref[...] = x * inv


def _finalize_body(s_ref, d_ref, o_ref):
    s = jnp.sum(s_ref[...], axis=0, keepdims=True)
    d = jnp.sum(d_ref[...], axis=0, keepdims=True)
    o_ref[...] = jnp.where(d > 0.0, s / jnp.maximum(d, 1.0), 1.0)


@functools.lru_cache(maxsize=None)
def _make_edge_kernel(n_nodes, d, ch, n_edges):
    nseg = d // LANES
    n_pad = -(-n_nodes // 2048) * 2048
    mesh = plsc.VectorSubcoreMesh(core_axis_name="c", subcore_axis_name="s")
    out_t = (
        jax.ShapeDtypeStruct((NUM_CORES, n_pad), jnp.float32),
        jax.ShapeDtypeStruct((NUM_CORES, n_pad), jnp.float32),
    )

    cp = pltpu.CompilerParams()
    if "needs_layout_passes" in pltpu.CompilerParams.__dataclass_fields__:
        cp = dataclasses.replace(cp, needs_layout_passes=False)

    @functools.partial(
        pl.kernel,
        out_type=out_t,
        mesh=mesh,
        compiler_params=cp,
        scratch_types=[
            pltpu.VMEM((ch, CHUNK), jnp.int32),
            pltpu.VMEM((ch, CHUNK), jnp.int32),
            pltpu.VMEM((CHUNK,), jnp.float32),
            pltpu.VMEM((2, CHUNK, d), jnp.float32),
            pltpu.VMEM((2, CHUNK, d), jnp.float32),
            pltpu.VMEM((CHUNK,), jnp.float32),
            pltpu.VMEM((2048,), jnp.float32),
            pltpu.VMEM_SHARED((n_pad,), jnp.float32),
            pltpu.VMEM_SHARED((n_pad,), jnp.float32),
            pltpu.SemaphoreType.DMA,
            pltpu.SemaphoreType.DMA,
        ],
    )
    def edge_kernel(xhat_hbm, src_hbm, dst_hbm, sums_hbm, degs_hbm,
                    src_v, dst_v, val_c, srows, drows, sim_v, stage_v,
                    shared_sum, shared_deg, sem_a, sem_b):
        cid = lax.axis_index("c")
        sid = lax.axis_index("s")
        wid = sid * NUM_CORES + cid
        zeros16 = jnp.zeros((LANES,), jnp.float32)
        lane_iota = lax.iota(jnp.int32, LANES)

        @pl.when(sid == 0)
        def _init():
            @pl.loop(0, 2048, step=LANES)
            def _z(i):
                stage_v[pl.ds(pl.multiple_of(i, LANES), LANES)] = zeros16

            @pl.loop(0, n_pad, step=2048)
            def _zs(i):
                ii = pl.multiple_of(i, 2048)
                pltpu.sync_copy(stage_v, shared_sum.at[pl.ds(ii, 2048)])
                pltpu.sync_copy(stage_v, shared_deg.at[pl.ds(ii, 2048)])

        pltpu.sync_copy(src_hbm.at[wid], src_v)
        pltpu.sync_copy(dst_hbm.at[wid], dst_v)

        pltpu.async_copy(xhat_hbm.at[src_v.at[0]], srows.at[0], sem_a)
        pltpu.async_copy(xhat_hbm.at[dst_v.at[0]], drows.at[0], sem_b)
        plsc.subcore_barrier()

        @pl.loop(0, ch)
        def _chunk(j):
            par = lax.rem(j, 2)
            srow = srows.at[par]
            drow = drows.at[par]
            pltpu.make_async_copy(xhat_hbm.at[src_v.at[j]], srow, sem_a).wait()
            pltpu.make_async_copy(xhat_hbm.at[dst_v.at[j]], drow, sem_b).wait()

            @pl.when(j + 1 < ch)
            def _prefetch():
                nxt = 1 - par
                pltpu.async_copy(xhat_hbm.at[src_v.at[j + 1]], srows.at[nxt], sem_a)
                pltpu.async_copy(xhat_hbm.at[dst_v.at[j + 1]], drows.at[nxt], sem_b)

            chunk_eid = (wid * ch + j) * CHUNK

            @pl.loop(0, CHUNK // LANES)
            def _group(g):
                base = pl.multiple_of(g * LANES, LANES)
                vmask = jnp.where(chunk_eid + base + lane_iota < n_edges,
                                  1.0, 0.0).astype(jnp.float32)
                sim_vec = zeros16
                for rr in range(LANES):
                    a = srow[base + rr, pl.ds(0, LANES)]
                    b = drow[base + rr, pl.ds(0, LANES)]
                    acc = a * b
                    for kk in range(1, nseg):
                        a = srow[base + rr, pl.ds(kk * LANES, LANES)]
                        b = drow[base + rr, pl.ds(kk * LANES, LANES)]
                        acc = acc + a * b
                    tot = jnp.sum(acc)
                    sim_vec = jnp.where(lane_iota == rr, tot, sim_vec)
                sim_v[pl.ds(base, LANES)] = sim_vec * vmask
                val_c[pl.ds(base, LANES)] = vmask

            pltpu.sync_copy(sim_v, shared_sum.at[src_v.at[j]], add=True)
            pltpu.sync_copy(sim_v, shared_sum.at[dst_v.at[j]], add=True)
            pltpu.sync_copy(val_c, shared_deg.at[src_v.at[j]], add=True)
            pltpu.sync_copy(val_c, shared_deg.at[dst_v.at[j]], add=True)

        plsc.subcore_barrier()

        @pl.when(sid == 0)
        def _drain():
            @pl.loop(0, n_pad, step=2048)
            def _d(i):
                ii = pl.multiple_of(i, 2048)
                pltpu.sync_copy(shared_sum.at[pl.ds(ii, 2048)], stage_v)
                pltpu.sync_copy(stage_v, sums_hbm.at[cid].at[pl.ds(ii, 2048)])
                pltpu.sync_copy(shared_deg.at[pl.ds(ii, 2048)], stage_v)
                pltpu.sync_copy(stage_v, degs_hbm.at[cid].at[pl.ds(ii, 2048)])

    return edge_kernel


def kernel(node_features, edge_index):
    n, d = node_features.shape
    e = edge_index.shape[1]

    xhat = pl.pallas_call(
        _normalize_body,
        out_shape=jax.ShapeDtypeStruct((n, d), jnp.float32),
    )(node_features)

    ch = -(-e // (NW * CHUNK))
    ep = NW * CHUNK * ch
    pad = ep - e
    src = edge_index[0].astype(jnp.int32)
    dst = edge_index[1].astype(jnp.int32)
    srcp = jnp.pad(src, (0, pad)).reshape(NW, ch, CHUNK)
    dstp = jnp.pad(dst, (0, pad)).reshape(NW, ch, CHUNK)

    sums, degs = _make_edge_kernel(n, d, ch, e)(xhat, srcp, dstp)

    n_pad = sums.shape[1]
    out = pl.pallas_call(
        _finalize_body,
        out_shape=jax.ShapeDtypeStruct((1, n_pad), jnp.float32),
    )(sums, degs)
    return out.reshape(n_pad)[:n]


# speedup vs baseline: 1.6502x; 1.0511x over previous
import dataclasses
import functools

import jax
import jax.numpy as jnp
from jax import lax
from jax.experimental import pallas as pl
from jax.experimental.pallas import tpu as pltpu
from jax.experimental.pallas import tpu_sc as plsc

EPS = 1e-8
LANES = 16
NUM_CORES = 2
NUM_SUBCORES = 16
NW = NUM_CORES * NUM_SUBCORES
CHUNK = 64


def _normalize_body(x_ref, o_ref):
    x = x_ref[...]
    ss = jnp.sum(x * x, axis=1, keepdims=True)
    inv = 1.0 / jnp.maximum(jnp.sqrt(ss), EPS)
    o_ref[...] = (x * inv).astype(jnp.bfloat16)


def _finalize_body(s_ref, d_ref, o_ref):
    s = jnp.sum(s_ref[...], axis=0, keepdims=True)
    d = jnp.sum(d_ref[...], axis=0, keepdims=True)
    o_ref[...] = jnp.where(d > 0.0, s / jnp.maximum(d, 1.0), 1.0)


@functools.lru_cache(maxsize=None)
def _make_edge_kernel(n_nodes, d, ch, n_edges):
    n_pad = -(-n_nodes // 2048) * 2048
    mesh = plsc.VectorSubcoreMesh(core_axis_name="c", subcore_axis_name="s")
    out_t = (
        jax.ShapeDtypeStruct((NUM_CORES, n_pad), jnp.float32),
        jax.ShapeDtypeStruct((NUM_CORES, n_pad), jnp.float32),
    )

    cp = pltpu.CompilerParams()
    if "needs_layout_passes" in pltpu.CompilerParams.__dataclass_fields__:
        cp = dataclasses.replace(cp, needs_layout_passes=False)

    @functools.partial(
        pl.kernel,
        out_type=out_t,
        mesh=mesh,
        compiler_params=cp,
        scratch_types=[
            pltpu.VMEM((ch, CHUNK), jnp.int32),
            pltpu.VMEM((ch, CHUNK), jnp.int32),
            pltpu.VMEM((CHUNK,), jnp.float32),
            pltpu.VMEM((2, CHUNK, d // 2), jnp.float32),
            pltpu.VMEM((2, CHUNK, d // 2), jnp.float32),
            pltpu.VMEM((CHUNK,), jnp.float32),
            pltpu.VMEM((2048,), jnp.float32),
            pltpu.VMEM_SHARED((n_pad,), jnp.float32),
            pltpu.VMEM_SHARED((n_pad,), jnp.float32),
            pltpu.SemaphoreType.DMA,
            pltpu.SemaphoreType.DMA,
        ],
    )
    def edge_kernel(xhat_hbm, src_hbm, dst_hbm, sums_hbm, degs_hbm,
                    src_v, dst_v, val_c, srows, drows, sim_v, stage_v,
                    shared_sum, shared_deg, sem_a, sem_b):
        cid = lax.axis_index("c")
        sid = lax.axis_index("s")
        wid = sid * NUM_CORES + cid
        zeros16 = jnp.zeros((LANES,), jnp.float32)
        lane_iota = lax.iota(jnp.int32, LANES)

        @pl.when(sid == 0)
        def _init():
            @pl.loop(0, 2048, step=LANES)
            def _z(i):
                stage_v[pl.ds(pl.multiple_of(i, LANES), LANES)] = zeros16

            @pl.loop(0, n_pad, step=2048)
            def _zs(i):
                ii = pl.multiple_of(i, 2048)
                pltpu.sync_copy(stage_v, shared_sum.at[pl.ds(ii, 2048)])
                pltpu.sync_copy(stage_v, shared_deg.at[pl.ds(ii, 2048)])

        pltpu.sync_copy(src_hbm.at[wid], src_v)
        pltpu.sync_copy(dst_hbm.at[wid], dst_v)

        pltpu.async_copy(xhat_hbm.at[src_v.at[0]], srows.at[0], sem_a)
        pltpu.async_copy(xhat_hbm.at[dst_v.at[0]], drows.at[0], sem_b)
        plsc.subcore_barrier()

        @pl.loop(0, ch)
        def _chunk(j):
            par = lax.rem(j, 2)
            srow = srows.at[par]
            drow = drows.at[par]
            pltpu.make_async_copy(xhat_hbm.at[src_v.at[j]], srow, sem_a).wait()
            pltpu.make_async_copy(xhat_hbm.at[dst_v.at[j]], drow, sem_b).wait()

            @pl.when(j + 1 < ch)
            def _prefetch():
                nxt = 1 - par
                pltpu.async_copy(xhat_hbm.at[src_v.at[j + 1]], srows.at[nxt], sem_a)
                pltpu.async_copy(xhat_hbm.at[dst_v.at[j + 1]], drows.at[nxt], sem_b)

            chunk_eid = (wid * ch + j) * CHUNK

            @pl.loop(0, CHUNK // LANES)
            def _group(g):
                base = pl.multiple_of(g * LANES, LANES)
                vmask = jnp.where(chunk_eid + base + lane_iota < n_edges,
                                  1.0, 0.0).astype(jnp.float32)
                sim_vec = zeros16
                for rr in range(LANES):
                    acc0 = zeros16
                    acc1 = zeros16
                    for ss in range(d // 32):
                        a = plsc.bitcast(
                            srow[base + rr, pl.ds(ss * LANES, LANES)],
                            jnp.bfloat16)
                        b = plsc.bitcast(
                            drow[base + rr, pl.ds(ss * LANES, LANES)],
                            jnp.bfloat16)
                        p0, p1 = plsc.unpack(
                            a * b, format=plsc.PackFormat.INTERLEAVED)
                        acc0 = acc0 + p0
                        acc1 = acc1 + p1
                    tot = jnp.sum(acc0 + acc1)
                    sim_vec = jnp.where(lane_iota == rr, tot, sim_vec)
                sim_v[pl.ds(base, LANES)] = sim_vec * vmask
                val_c[pl.ds(base, LANES)] = vmask

            pltpu.sync_copy(sim_v, shared_sum.at[src_v.at[j]], add=True)
            pltpu.sync_copy(sim_v, shared_sum.at[dst_v.at[j]], add=True)
            pltpu.sync_copy(val_c, shared_deg.at[src_v.at[j]], add=True)
            pltpu.sync_copy(val_c, shared_deg.at[dst_v.at[j]], add=True)

        plsc.subcore_barrier()

        @pl.when(sid == 0)
        def _drain():
            @pl.loop(0, n_pad, step=2048)
            def _d(i):
                ii = pl.multiple_of(i, 2048)
                pltpu.sync_copy(shared_sum.at[pl.ds(ii, 2048)], stage_v)
                pltpu.sync_copy(stage_v, sums_hbm.at[cid].at[pl.ds(ii, 2048)])
                pltpu.sync_copy(shared_deg.at[pl.ds(ii, 2048)], stage_v)
                pltpu.sync_copy(stage_v, degs_hbm.at[cid].at[pl.ds(ii, 2048)])

    return edge_kernel


def kernel(node_features, edge_index):
    n, d = node_features.shape
    e = edge_index.shape[1]

    xhat = pl.pallas_call(
        _normalize_body,
        out_shape=jax.ShapeDtypeStruct((n, d), jnp.bfloat16),
    )(node_features)
    xhat = jax.lax.bitcast_convert_type(
        xhat.reshape(n, d // 2, 2), jnp.float32)

    ch = -(-e // (NW * CHUNK))
    ep = NW * CHUNK * ch
    pad = ep - e
    src = edge_index[0].astype(jnp.int32)
    dst = edge_index[1].astype(jnp.int32)
    srcp = jnp.pad(src, (0, pad)).reshape(NW, ch, CHUNK)
    dstp = jnp.pad(dst, (0, pad)).reshape(NW, ch, CHUNK)

    sums, degs = _make_edge_kernel(n, d, ch, e)(xhat, srcp, dstp)

    n_pad = sums.shape[1]
    out = pl.pallas_call(
        _finalize_body,
        out_shape=jax.ShapeDtypeStruct((1, n_pad), jnp.float32),
    )(sums, degs)
    return out.reshape(n_pad)[:n]


# speedup vs baseline: 2.4568x; 1.4887x over previous
import dataclasses
import functools

import jax
import jax.numpy as jnp
from jax import lax
from jax.experimental import pallas as pl
from jax.experimental.pallas import tpu as pltpu
from jax.experimental.pallas import tpu_sc as plsc

EPS = 1e-8
LANES = 16
NUM_CORES = 2
NUM_SUBCORES = 16
NW = NUM_CORES * NUM_SUBCORES
CHUNK = 64


def _normalize_body(x_ref, o_ref):
    x = x_ref[...]
    h = x.shape[1] // 2
    ss = jnp.sum(x * x, axis=1, keepdims=True)
    inv = 1.0 / jnp.maximum(jnp.sqrt(ss), EPS)
    xh = x * inv
    lo = lax.bitcast_convert_type(
        xh[:, :h].astype(jnp.bfloat16), jnp.uint16).astype(jnp.uint32)
    hi = lax.bitcast_convert_type(
        xh[:, h:].astype(jnp.bfloat16), jnp.uint16).astype(jnp.uint32)
    o_ref[...] = lax.bitcast_convert_type(lo | (hi << 16), jnp.float32)


def _finalize_body(s_ref, d_ref, o_ref):
    s = jnp.sum(s_ref[...], axis=0, keepdims=True)
    d = jnp.sum(d_ref[...], axis=0, keepdims=True)
    o_ref[...] = jnp.where(d > 0.0, s / jnp.maximum(d, 1.0), 1.0)


@functools.lru_cache(maxsize=None)
def _make_edge_kernel(n_nodes, d, ch, n_edges):
    n_pad = -(-n_nodes // 2048) * 2048
    mesh = plsc.VectorSubcoreMesh(core_axis_name="c", subcore_axis_name="s")
    out_t = (
        jax.ShapeDtypeStruct((NUM_CORES, n_pad), jnp.float32),
        jax.ShapeDtypeStruct((NUM_CORES, n_pad), jnp.float32),
    )

    cp = pltpu.CompilerParams()
    if "needs_layout_passes" in pltpu.CompilerParams.__dataclass_fields__:
        cp = dataclasses.replace(cp, needs_layout_passes=False)

    @functools.partial(
        pl.kernel,
        out_type=out_t,
        mesh=mesh,
        compiler_params=cp,
        scratch_types=[
            pltpu.VMEM((ch, CHUNK), jnp.int32),
            pltpu.VMEM((ch, CHUNK), jnp.int32),
            pltpu.VMEM((CHUNK,), jnp.float32),
            pltpu.VMEM((2, CHUNK, d // 2), jnp.float32),
            pltpu.VMEM((2, CHUNK, d // 2), jnp.float32),
            pltpu.VMEM((CHUNK,), jnp.float32),
            pltpu.VMEM((2048,), jnp.float32),
            pltpu.VMEM_SHARED((n_pad,), jnp.float32),
            pltpu.VMEM_SHARED((n_pad,), jnp.float32),
            pltpu.SemaphoreType.DMA,
            pltpu.SemaphoreType.DMA,
        ],
    )
    def edge_kernel(xhat_hbm, src_hbm, dst_hbm, sums_hbm, degs_hbm,
                    src_v, dst_v, val_c, srows, drows, sim_v, stage_v,
                    shared_sum, shared_deg, sem_a, sem_b):
        cid = lax.axis_index("c")
        sid = lax.axis_index("s")
        wid = sid * NUM_CORES + cid
        zeros16 = jnp.zeros((LANES,), jnp.float32)
        lane_iota = lax.iota(jnp.int32, LANES)

        @pl.when(sid == 0)
        def _init():
            @pl.loop(0, 2048, step=LANES)
            def _z(i):
                stage_v[pl.ds(pl.multiple_of(i, LANES), LANES)] = zeros16

            @pl.loop(0, n_pad, step=2048)
            def _zs(i):
                ii = pl.multiple_of(i, 2048)
                pltpu.sync_copy(stage_v, shared_sum.at[pl.ds(ii, 2048)])
                pltpu.sync_copy(stage_v, shared_deg.at[pl.ds(ii, 2048)])

        pltpu.sync_copy(src_hbm.at[wid], src_v)
        pltpu.sync_copy(dst_hbm.at[wid], dst_v)

        pltpu.async_copy(xhat_hbm.at[src_v.at[0]], srows.at[0], sem_a)
        pltpu.async_copy(xhat_hbm.at[dst_v.at[0]], drows.at[0], sem_b)
        plsc.subcore_barrier()

        @pl.loop(0, ch)
        def _chunk(j):
            par = lax.rem(j, 2)
            srow = srows.at[par]
            drow = drows.at[par]
            pltpu.make_async_copy(xhat_hbm.at[src_v.at[j]], srow, sem_a).wait()
            pltpu.make_async_copy(xhat_hbm.at[dst_v.at[j]], drow, sem_b).wait()

            @pl.when(j + 1 < ch)
            def _prefetch():
                nxt = 1 - par
                pltpu.async_copy(xhat_hbm.at[src_v.at[j + 1]], srows.at[nxt], sem_a)
                pltpu.async_copy(xhat_hbm.at[dst_v.at[j + 1]], drows.at[nxt], sem_b)

            chunk_eid = (wid * ch + j) * CHUNK

            @pl.loop(0, CHUNK // LANES)
            def _group(g):
                base = pl.multiple_of(g * LANES, LANES)
                vmask = jnp.where(chunk_eid + base + lane_iota < n_edges,
                                  1.0, 0.0).astype(jnp.float32)
                sim_vec = zeros16
                for rr in range(LANES):
                    acc0 = zeros16
                    acc1 = zeros16
                    for ss in range(d // 32):
                        a = plsc.bitcast(
                            srow[base + rr, pl.ds(ss * LANES, LANES)],
                            jnp.bfloat16)
                        b = plsc.bitcast(
                            drow[base + rr, pl.ds(ss * LANES, LANES)],
                            jnp.bfloat16)
                        p0, p1 = plsc.unpack(
                            a * b, format=plsc.PackFormat.INTERLEAVED)
                        acc0 = acc0 + p0
                        acc1 = acc1 + p1
                    tot = jnp.sum(acc0 + acc1)
                    sim_vec = jnp.where(lane_iota == rr, tot, sim_vec)
                sim_v[pl.ds(base, LANES)] = sim_vec * vmask
                val_c[pl.ds(base, LANES)] = vmask

            pltpu.sync_copy(sim_v, shared_sum.at[src_v.at[j]], add=True)
            pltpu.sync_copy(sim_v, shared_sum.at[dst_v.at[j]], add=True)
            pltpu.sync_copy(val_c, shared_deg.at[src_v.at[j]], add=True)
            pltpu.sync_copy(val_c, shared_deg.at[dst_v.at[j]], add=True)

        plsc.subcore_barrier()

        @pl.when(sid == 0)
        def _drain():
            @pl.loop(0, n_pad, step=2048)
            def _d(i):
                ii = pl.multiple_of(i, 2048)
                pltpu.sync_copy(shared_sum.at[pl.ds(ii, 2048)], stage_v)
                pltpu.sync_copy(stage_v, sums_hbm.at[cid].at[pl.ds(ii, 2048)])
                pltpu.sync_copy(shared_deg.at[pl.ds(ii, 2048)], stage_v)
                pltpu.sync_copy(stage_v, degs_hbm.at[cid].at[pl.ds(ii, 2048)])

    return edge_kernel


def kernel(node_features, edge_index):
    n, d = node_features.shape
    e = edge_index.shape[1]

    xhat = pl.pallas_call(
        _normalize_body,
        out_shape=jax.ShapeDtypeStruct((n, d // 2), jnp.float32),
    )(node_features)

    ch = -(-e // (NW * CHUNK))
    ep = NW * CHUNK * ch
    pad = ep - e
    src = edge_index[0].astype(jnp.int32)
    dst = edge_index[1].astype(jnp.int32)
    srcp = jnp.pad(src, (0, pad)).reshape(NW, ch, CHUNK)
    dstp = jnp.pad(dst, (0, pad)).reshape(NW, ch, CHUNK)

    sums, degs = _make_edge_kernel(n, d, ch, e)(xhat, srcp, dstp)

    n_pad = sums.shape[1]
    out = pl.pallas_call(
        _finalize_body,
        out_shape=jax.ShapeDtypeStruct((1, n_pad), jnp.float32),
    )(sums, degs)
    return out.reshape(n_pad)[:n]


# speedup vs baseline: 2.4602x; 1.0014x over previous
import dataclasses
import functools

import jax
import jax.numpy as jnp
from jax import lax
from jax.experimental import pallas as pl
from jax.experimental.pallas import tpu as pltpu
from jax.experimental.pallas import tpu_sc as plsc

EPS = 1e-8
LANES = 16
NUM_CORES = 2
NUM_SUBCORES = 16
NW = NUM_CORES * NUM_SUBCORES
CHUNK = 64


def _normalize_body(x_ref, o_ref):
    x = x_ref[...]
    h = x.shape[1] // 2
    ss = jnp.sum(x * x, axis=1, keepdims=True)
    inv = 1.0 / jnp.maximum(jnp.sqrt(ss), EPS)
    xh = x * inv
    lo = lax.bitcast_convert_type(
        xh[:, :h].astype(jnp.bfloat16), jnp.uint16).astype(jnp.uint32)
    hi = lax.bitcast_convert_type(
        xh[:, h:].astype(jnp.bfloat16), jnp.uint16).astype(jnp.uint32)
    o_ref[...] = lax.bitcast_convert_type(lo | (hi << 16), jnp.float32)


def _finalize_body(s_ref, d_ref, o_ref):
    s = jnp.sum(s_ref[...], axis=0, keepdims=True)
    d = jnp.sum(d_ref[...], axis=0, keepdims=True)
    o_ref[...] = jnp.where(d > 0.0, s / jnp.maximum(d, 1.0), 1.0)


@functools.lru_cache(maxsize=None)
def _make_edge_kernel(n_nodes, d, ch, n_edges):
    n_pad = -(-n_nodes // 2048) * 2048
    mesh = plsc.VectorSubcoreMesh(core_axis_name="c", subcore_axis_name="s")
    out_t = (
        jax.ShapeDtypeStruct((NUM_CORES, n_pad), jnp.float32),
        jax.ShapeDtypeStruct((NUM_CORES, n_pad), jnp.float32),
    )

    cp = pltpu.CompilerParams()
    if "needs_layout_passes" in pltpu.CompilerParams.__dataclass_fields__:
        cp = dataclasses.replace(cp, needs_layout_passes=False)

    @functools.partial(
        pl.kernel,
        out_type=out_t,
        mesh=mesh,
        compiler_params=cp,
        scratch_types=[
            pltpu.VMEM((ch, CHUNK), jnp.int32),
            pltpu.VMEM((ch, CHUNK), jnp.int32),
            pltpu.VMEM((2, CHUNK), jnp.float32),
            pltpu.VMEM((2, CHUNK, d // 2), jnp.float32),
            pltpu.VMEM((2, CHUNK, d // 2), jnp.float32),
            pltpu.VMEM((2, CHUNK), jnp.float32),
            pltpu.VMEM((2048,), jnp.float32),
            pltpu.VMEM_SHARED((n_pad,), jnp.float32),
            pltpu.VMEM_SHARED((n_pad,), jnp.float32),
            pltpu.SemaphoreType.DMA,
            pltpu.SemaphoreType.DMA,
            pltpu.SemaphoreType.DMA,
        ],
    )
    def edge_kernel(xhat_hbm, src_hbm, dst_hbm, sums_hbm, degs_hbm,
                    src_v, dst_v, val_c, srows, drows, sim_v, stage_v,
                    shared_sum, shared_deg, sem_a, sem_b, sem_s):
        cid = lax.axis_index("c")
        sid = lax.axis_index("s")
        wid = sid * NUM_CORES + cid
        zeros16 = jnp.zeros((LANES,), jnp.float32)
        lane_iota = lax.iota(jnp.int32, LANES)

        @pl.when(sid == 0)
        def _init():
            @pl.loop(0, 2048, step=LANES)
            def _z(i):
                stage_v[pl.ds(pl.multiple_of(i, LANES), LANES)] = zeros16

            @pl.loop(0, n_pad, step=2048)
            def _zs(i):
                ii = pl.multiple_of(i, 2048)
                pltpu.sync_copy(stage_v, shared_sum.at[pl.ds(ii, 2048)])
                pltpu.sync_copy(stage_v, shared_deg.at[pl.ds(ii, 2048)])

        pltpu.sync_copy(src_hbm.at[wid], src_v)
        pltpu.sync_copy(dst_hbm.at[wid], dst_v)

        pltpu.async_copy(xhat_hbm.at[src_v.at[0]], srows.at[0], sem_a)
        pltpu.async_copy(xhat_hbm.at[dst_v.at[0]], drows.at[0], sem_b)
        plsc.subcore_barrier()

        @pl.loop(0, ch)
        def _chunk(j):
            par = lax.rem(j, 2)
            srow = srows.at[par]
            drow = drows.at[par]
            pltpu.make_async_copy(xhat_hbm.at[src_v.at[j]], srow, sem_a).wait()
            pltpu.make_async_copy(xhat_hbm.at[dst_v.at[j]], drow, sem_b).wait()

            @pl.when(j + 1 < ch)
            def _prefetch():
                nxt = 1 - par
                pltpu.async_copy(xhat_hbm.at[src_v.at[j + 1]], srows.at[nxt], sem_a)
                pltpu.async_copy(xhat_hbm.at[dst_v.at[j + 1]], drows.at[nxt], sem_b)

            @pl.when(j >= 2)
            def _drain_scatters():
                jm = j - 2
                pltpu.make_async_copy(
                    sim_v.at[par], shared_sum.at[src_v.at[jm]], sem_s).wait()
                pltpu.make_async_copy(
                    sim_v.at[par], shared_sum.at[dst_v.at[jm]], sem_s).wait()
                pltpu.make_async_copy(
                    val_c.at[par], shared_deg.at[src_v.at[jm]], sem_s).wait()
                pltpu.make_async_copy(
                    val_c.at[par], shared_deg.at[dst_v.at[jm]], sem_s).wait()

            chunk_eid = (wid * ch + j) * CHUNK

            @pl.loop(0, CHUNK // LANES)
            def _group(g):
                base = pl.multiple_of(g * LANES, LANES)
                vmask = jnp.where(chunk_eid + base + lane_iota < n_edges,
                                  1.0, 0.0).astype(jnp.float32)
                sim_vec = zeros16
                for rr in range(LANES):
                    acc0 = zeros16
                    acc1 = zeros16
                    for ss in range(d // 32):
                        a = plsc.bitcast(
                            srow[base + rr, pl.ds(ss * LANES, LANES)],
                            jnp.bfloat16)
                        b = plsc.bitcast(
                            drow[base + rr, pl.ds(ss * LANES, LANES)],
                            jnp.bfloat16)
                        p0, p1 = plsc.unpack(
                            a * b, format=plsc.PackFormat.INTERLEAVED)
                        acc0 = acc0 + p0
                        acc1 = acc1 + p1
                    tot = jnp.sum(acc0 + acc1)
                    sim_vec = jnp.where(lane_iota == rr, tot, sim_vec)
                sim_v[par, pl.ds(base, LANES)] = sim_vec * vmask
                val_c[par, pl.ds(base, LANES)] = vmask

            pltpu.async_copy(sim_v.at[par], shared_sum.at[src_v.at[j]],
                             sem_s, add=True)
            pltpu.async_copy(sim_v.at[par], shared_sum.at[dst_v.at[j]],
                             sem_s, add=True)
            pltpu.async_copy(val_c.at[par], shared_deg.at[src_v.at[j]],
                             sem_s, add=True)
            pltpu.async_copy(val_c.at[par], shared_deg.at[dst_v.at[j]],
                             sem_s, add=True)

        for jt in range(max(ch - 2, 0), ch):
            pt = jt % 2
            pltpu.make_async_copy(
                sim_v.at[pt], shared_sum.at[src_v.at[jt]], sem_s).wait()
            pltpu.make_async_copy(
                sim_v.at[pt], shared_sum.at[dst_v.at[jt]], sem_s).wait()
            pltpu.make_async_copy(
                val_c.at[pt], shared_deg.at[src_v.at[jt]], sem_s).wait()
            pltpu.make_async_copy(
                val_c.at[pt], shared_deg.at[dst_v.at[jt]], sem_s).wait()

        plsc.subcore_barrier()

        @pl.when(sid == 0)
        def _drain():
            @pl.loop(0, n_pad, step=2048)
            def _d(i):
                ii = pl.multiple_of(i, 2048)
                pltpu.sync_copy(shared_sum.at[pl.ds(ii, 2048)], stage_v)
                pltpu.sync_copy(stage_v, sums_hbm.at[cid].at[pl.ds(ii, 2048)])
                pltpu.sync_copy(shared_deg.at[pl.ds(ii, 2048)], stage_v)
                pltpu.sync_copy(stage_v, degs_hbm.at[cid].at[pl.ds(ii, 2048)])

    return edge_kernel


def kernel(node_features, edge_index):
    n, d = node_features.shape
    e = edge_index.shape[1]

    xhat = pl.pallas_call(
        _normalize_body,
        out_shape=jax.ShapeDtypeStruct((n, d // 2), jnp.float32),
    )(node_features)

    ch = -(-e // (NW * CHUNK))
    ep = NW * CHUNK * ch
    pad = ep - e
    src = edge_index[0].astype(jnp.int32)
    dst = edge_index[1].astype(jnp.int32)
    srcp = jnp.pad(src, (0, pad)).reshape(NW, ch, CHUNK)
    dstp = jnp.pad(dst, (0, pad)).reshape(NW, ch, CHUNK)

    sums, degs = _make_edge_kernel(n, d, ch, e)(xhat, srcp, dstp)

    n_pad = sums.shape[1]
    out = pl.pallas_call(
        _finalize_body,
        out_shape=jax.ShapeDtypeStruct((1, n_pad), jnp.float32),
    )(sums, degs)
    return out.reshape(n_pad)[:n]


# speedup vs baseline: 2.5621x; 1.0414x over previous
import dataclasses
import functools

import jax
import jax.numpy as jnp
from jax import lax
from jax.experimental import pallas as pl
from jax.experimental.pallas import tpu as pltpu
from jax.experimental.pallas import tpu_sc as plsc

EPS = 1e-8
LANES = 16
NUM_CORES = 2
NUM_SUBCORES = 16
NW = NUM_CORES * NUM_SUBCORES
CHUNK = 64
NSLOT = 4
PF = 3


def _normalize_body(x_ref, o_ref):
    x = x_ref[...]
    h = x.shape[1] // 2
    ss = jnp.sum(x * x, axis=1, keepdims=True)
    inv = 1.0 / jnp.maximum(jnp.sqrt(ss), EPS)
    xh = x * inv
    lo = lax.bitcast_convert_type(
        xh[:, :h].astype(jnp.bfloat16), jnp.uint16).astype(jnp.uint32)
    hi = lax.bitcast_convert_type(
        xh[:, h:].astype(jnp.bfloat16), jnp.uint16).astype(jnp.uint32)
    o_ref[...] = lax.bitcast_convert_type(lo | (hi << 16), jnp.float32)


def _finalize_body(s_ref, d_ref, o_ref):
    s = jnp.sum(s_ref[...], axis=0, keepdims=True)
    d = jnp.sum(d_ref[...], axis=0, keepdims=True)
    o_ref[...] = jnp.where(d > 0.0, s / jnp.maximum(d, 1.0), 1.0)


@functools.lru_cache(maxsize=None)
def _make_edge_kernel(n_nodes, d, ch, n_edges):
    n_pad = -(-n_nodes // 2048) * 2048
    dw = d // 2
    mesh = plsc.VectorSubcoreMesh(core_axis_name="c", subcore_axis_name="s")
    out_t = (
        jax.ShapeDtypeStruct((NUM_CORES, n_pad), jnp.float32),
        jax.ShapeDtypeStruct((NUM_CORES, n_pad), jnp.float32),
    )

    cp = pltpu.CompilerParams()
    if "needs_layout_passes" in pltpu.CompilerParams.__dataclass_fields__:
        cp = dataclasses.replace(cp, needs_layout_passes=False)

    @functools.partial(
        pl.kernel,
        out_type=out_t,
        mesh=mesh,
        compiler_params=cp,
        scratch_types=[
            pltpu.VMEM((ch, 2 * CHUNK), jnp.int32),
            pltpu.VMEM((ch, CHUNK), jnp.int32),
            pltpu.VMEM((ch, CHUNK), jnp.int32),
            pltpu.VMEM((2, CHUNK), jnp.float32),
            pltpu.VMEM((NSLOT, 2 * CHUNK, dw), jnp.float32),
            pltpu.VMEM((2, CHUNK), jnp.float32),
            pltpu.VMEM((2048,), jnp.float32),
            pltpu.VMEM_SHARED((n_pad,), jnp.float32),
            pltpu.VMEM_SHARED((n_pad,), jnp.float32),
            pltpu.SemaphoreType.DMA,
            pltpu.SemaphoreType.DMA,
        ],
    )
    def edge_kernel(xhat_hbm, comb_hbm, src_hbm, dst_hbm, sums_hbm, degs_hbm,
                    comb_v, src_v, dst_v, val_c, rows, sim_v, stage_v,
                    shared_sum, shared_deg, sem_a, sem_s):
        cid = lax.axis_index("c")
        sid = lax.axis_index("s")
        wid = sid * NUM_CORES + cid
        zeros16 = jnp.zeros((LANES,), jnp.float32)
        lane_iota = lax.iota(jnp.int32, LANES)

        @pl.when(sid == 0)
        def _init():
            @pl.loop(0, 2048, step=LANES)
            def _z(i):
                stage_v[pl.ds(pl.multiple_of(i, LANES), LANES)] = zeros16

            @pl.loop(0, n_pad, step=2048)
            def _zs(i):
                ii = pl.multiple_of(i, 2048)
                pltpu.sync_copy(stage_v, shared_sum.at[pl.ds(ii, 2048)])
                pltpu.sync_copy(stage_v, shared_deg.at[pl.ds(ii, 2048)])

        pltpu.sync_copy(comb_hbm.at[wid], comb_v)
        pltpu.sync_copy(src_hbm.at[wid], src_v)
        pltpu.sync_copy(dst_hbm.at[wid], dst_v)

        for jw in range(min(PF, ch)):
            pltpu.async_copy(xhat_hbm.at[comb_v.at[jw]], rows.at[jw % NSLOT],
                             sem_a)
        plsc.subcore_barrier()

        @pl.loop(0, ch)
        def _chunk(j):
            slot = lax.rem(j, NSLOT)
            par = lax.rem(j, 2)
            rbuf = rows.at[slot]
            pltpu.make_async_copy(xhat_hbm.at[comb_v.at[j]], rbuf, sem_a).wait()

            @pl.when(j + PF < ch)
            def _prefetch():
                nxt = lax.rem(j + PF, NSLOT)
                pltpu.async_copy(xhat_hbm.at[comb_v.at[j + PF]],
                                 rows.at[nxt], sem_a)

            @pl.when(j >= 2)
            def _drain_scatters():
                jm = j - 2
                pltpu.make_async_copy(
                    sim_v.at[par], shared_sum.at[src_v.at[jm]], sem_s).wait()
                pltpu.make_async_copy(
                    sim_v.at[par], shared_sum.at[dst_v.at[jm]], sem_s).wait()
                pltpu.make_async_copy(
                    val_c.at[par], shared_deg.at[src_v.at[jm]], sem_s).wait()
                pltpu.make_async_copy(
                    val_c.at[par], shared_deg.at[dst_v.at[jm]], sem_s).wait()

            chunk_eid = (wid * ch + j) * CHUNK

            @pl.loop(0, CHUNK // LANES)
            def _group(g):
                base = pl.multiple_of(g * LANES, LANES)
                vmask = jnp.where(chunk_eid + base + lane_iota < n_edges,
                                  1.0, 0.0).astype(jnp.float32)
                sim_vec = zeros16
                for rr in range(LANES):
                    acc0 = zeros16
                    acc1 = zeros16
                    for ss in range(dw // LANES):
                        a = plsc.bitcast(
                            rbuf[base + rr, pl.ds(ss * LANES, LANES)],
                            jnp.bfloat16)
                        b = plsc.bitcast(
                            rbuf[CHUNK + base + rr, pl.ds(ss * LANES, LANES)],
                            jnp.bfloat16)
                        p0, p1 = plsc.unpack(
                            a * b, format=plsc.PackFormat.INTERLEAVED)
                        acc0 = acc0 + p0
                        acc1 = acc1 + p1
                    tot = jnp.sum(acc0 + acc1)
                    sim_vec = jnp.where(lane_iota == rr, tot, sim_vec)
                sim_v[par, pl.ds(base, LANES)] = sim_vec * vmask
                val_c[par, pl.ds(base, LANES)] = vmask

            pltpu.async_copy(sim_v.at[par], shared_sum.at[src_v.at[j]],
                             sem_s, add=True)
            pltpu.async_copy(sim_v.at[par], shared_sum.at[dst_v.at[j]],
                             sem_s, add=True)
            pltpu.async_copy(val_c.at[par], shared_deg.at[src_v.at[j]],
                             sem_s, add=True)
            pltpu.async_copy(val_c.at[par], shared_deg.at[dst_v.at[j]],
                             sem_s, add=True)

        for jt in range(max(ch - 2, 0), ch):
            pt = jt % 2
            pltpu.make_async_copy(
                sim_v.at[pt], shared_sum.at[src_v.at[jt]], sem_s).wait()
            pltpu.make_async_copy(
                sim_v.at[pt], shared_sum.at[dst_v.at[jt]], sem_s).wait()
            pltpu.make_async_copy(
                val_c.at[pt], shared_deg.at[src_v.at[jt]], sem_s).wait()
            pltpu.make_async_copy(
                val_c.at[pt], shared_deg.at[dst_v.at[jt]], sem_s).wait()

        plsc.subcore_barrier()

        @pl.when(sid == 0)
        def _drain():
            @pl.loop(0, n_pad, step=2048)
            def _d(i):
                ii = pl.multiple_of(i, 2048)
                pltpu.sync_copy(shared_sum.at[pl.ds(ii, 2048)], stage_v)
                pltpu.sync_copy(stage_v, sums_hbm.at[cid].at[pl.ds(ii, 2048)])
                pltpu.sync_copy(shared_deg.at[pl.ds(ii, 2048)], stage_v)
                pltpu.sync_copy(stage_v, degs_hbm.at[cid].at[pl.ds(ii, 2048)])

    return edge_kernel


def kernel(node_features, edge_index):
    n, d = node_features.shape
    e = edge_index.shape[1]

    xhat = pl.pallas_call(
        _normalize_body,
        out_shape=jax.ShapeDtypeStruct((n, d // 2), jnp.float32),
    )(node_features)

    ch = -(-e // (NW * CHUNK))
    ep = NW * CHUNK * ch
    pad = ep - e
    src = edge_index[0].astype(jnp.int32)
    dst = edge_index[1].astype(jnp.int32)
    srcp = jnp.pad(src, (0, pad)).reshape(NW, ch, CHUNK)
    dstp = jnp.pad(dst, (0, pad)).reshape(NW, ch, CHUNK)
    comb = jnp.concatenate([srcp, dstp], axis=-1)

    sums, degs = _make_edge_kernel(n, d, ch, e)(xhat, comb, srcp, dstp)

    n_pad = sums.shape[1]
    out = pl.pallas_call(
        _finalize_body,
        out_shape=jax.ShapeDtypeStruct((1, n_pad), jnp.float32),
    )(sums, degs)
    return out.reshape(n_pad)[:n]
